# baseline (speedup 1.0000x reference)
import jax
import jax.numpy as jnp
from jax import lax
from jax.experimental import pallas as pl
from jax.experimental.pallas import tpu as pltpu
from jax.experimental.pallas import tpu_sc as plsc

B = 16384
D = 64
NE = 1000000
NR = 1000
L = 16
NC = 2
NS = 16
NW = NC * NS
BPW = B // NW
C = 16
NCHUNK = BPW // C


def _transe_body(e1_hbm, e2_hbm, rlb_hbm, ent_hbm, relf_hbm, gam_hbm,
                 out_hbm,
                 e1_v, e2_v, rlb_v, rel_v, head_v, tail_v, out_v, gam_v,
                 sem1, sem2, sem3):
    wid = lax.axis_index("s") * NC + lax.axis_index("c")
    base = wid * BPW

    pltpu.sync_copy(e1_hbm.at[pl.ds(base, BPW)], e1_v)
    pltpu.sync_copy(e2_hbm.at[pl.ds(base, BPW)], e2_v)
    pltpu.sync_copy(rlb_hbm.at[pl.ds(base, BPW)], rlb_v)
    pltpu.sync_copy(gam_hbm, gam_v)
    pltpu.sync_copy(relf_hbm, rel_v)

    gam = gam_v[...]
    j = lax.iota(jnp.int32, L)

    def chunk_body(k, carry):
        off = k * C
        t1v = e1_v[pl.ds(off, L)]
        t2v = e2_v[pl.ds(off, L)]
        copies = []
        for jj in range(C):
            copies.append(pltpu.async_copy(
                ent_hbm.at[pl.ds(t1v[jj], 1), :], head_v.at[pl.ds(jj, 1), :],
                sem1))
            copies.append(pltpu.async_copy(
                ent_hbm.at[pl.ds(t2v[jj], 1), :], tail_v.at[pl.ds(jj, 1), :],
                sem2))
        for cp in copies:
            cp.wait()

        rb = rlb_v[pl.ds(off, L)]
        acc = jnp.zeros((L,), jnp.float32)
        for d in range(D):
            dcol = jnp.full((L,), d, jnp.int32)
            h = plsc.load_gather(head_v, [j, dcol])
            t = plsc.load_gather(tail_v, [j, dcol])
            r = plsc.load_gather(rel_v, [rb + d])
            acc = acc + jnp.abs(h + r - t)
        score = gam - acc
        out_v[pl.ds(off, L)] = 1.0 / (1.0 + jnp.exp(-score))
        return carry

    lax.fori_loop(0, NCHUNK, chunk_body, 0)

    pltpu.sync_copy(out_v, out_hbm.at[pl.ds(base, BPW)])


@jax.jit
def _transe_call(e1, e2, rlb, ent, relf, gam_vec):
    mesh = plsc.VectorSubcoreMesh(core_axis_name="c", subcore_axis_name="s")
    f = pl.kernel(
        _transe_body,
        mesh=mesh,
        compiler_params=pltpu.CompilerParams(needs_layout_passes=False),
        out_type=jax.ShapeDtypeStruct((B,), jnp.float32),
        scratch_types=[
            pltpu.VMEM((BPW,), jnp.int32),
            pltpu.VMEM((BPW,), jnp.int32),
            pltpu.VMEM((BPW,), jnp.int32),
            pltpu.VMEM((NR * D,), jnp.float32),
            pltpu.VMEM((C, D), jnp.float32),
            pltpu.VMEM((C, D), jnp.float32),
            pltpu.VMEM((BPW,), jnp.float32),
            pltpu.VMEM((L,), jnp.float32),
            pltpu.SemaphoreType.DMA,
            pltpu.SemaphoreType.DMA,
            pltpu.SemaphoreType.DMA,
        ],
    )
    return f(e1, e2, rlb, ent, relf, gam_vec)


def kernel(e1_idx, rel_idx, e2_idx, emb_ent_real, emb_rel_real, gamma):
    e1 = e1_idx.astype(jnp.int32)
    e2 = e2_idx.astype(jnp.int32)
    rlb = rel_idx.astype(jnp.int32) * D
    relf = emb_rel_real.reshape(NR * D)
    gam_vec = jnp.full((L,), gamma, jnp.float32)
    return _transe_call(e1, e2, rlb, emb_ent_real, relf, gam_vec)
